# baseline (speedup 1.0000x reference)
import functools

import jax
import jax.numpy as jnp
from jax import lax
from jax.experimental import pallas as pl
from jax.experimental.pallas import tpu as pltpu
from jax.experimental.pallas import tpu_sc as plsc

N = 100000
NC = 2
NS = 16
NW = NC * NS

ROWS_FULL = 3136
ROWS_LAST = N - (NW - 1) * ROWS_FULL
ITERS_FULL = ROWS_FULL // 16
ITERS_LAST = ROWS_LAST // 16


def _sc_loss_kernel(lf_hbm, lab_hbm, w_hbm, out_hbm,
                    lvm, labvm, wvm, partvm, allvm, outvm, shared):
    c = lax.axis_index("c")
    s = lax.axis_index("s")
    wid = s * NC + c

    base_rows = wid * ROWS_FULL

    @pl.when(wid != NW - 1)
    def _():
        pltpu.sync_copy(lab_hbm.at[pl.ds(base_rows, ROWS_FULL)], labvm)
        pltpu.sync_copy(lf_hbm.at[pl.ds(2 * base_rows, 2 * ROWS_FULL)], lvm)

    @pl.when(wid == NW - 1)
    def _():
        pltpu.sync_copy(lab_hbm.at[pl.ds(base_rows, ROWS_LAST)],
                        labvm.at[pl.ds(0, ROWS_LAST)])
        pltpu.sync_copy(lf_hbm.at[pl.ds(2 * base_rows, 2 * ROWS_LAST)],
                        lvm.at[pl.ds(0, 2 * ROWS_LAST)])

    pltpu.sync_copy(w_hbm, wvm)

    iota = lax.iota(jnp.int32, 16)
    idx2 = iota * 2

    def body(i, carry):
        num, den = carry
        rowoff = i * 16
        lab = labvm[pl.ds(rowoff, 16)]
        idx0 = idx2 + rowoff * 2
        idx1 = idx0 + 1
        l0 = plsc.load_gather(lvm, [idx0])
        l1 = plsc.load_gather(lvm, [idx1])
        yf = lab.astype(jnp.float32)
        z = (2.0 * yf - 1.0) * (l0 - l1)
        a = jnp.abs(z)
        e = jnp.exp(-a)
        t = e / (e + 2.0)
        u = t * t
        l1p = t * (2.0 + u * (2.0 / 3.0 + u * (2.0 / 5.0 + u * (2.0 / 7.0))))
        nll = jnp.maximum(z, 0.0) + l1p
        wv = plsc.load_gather(wvm, [lab])
        return num + wv * nll, den + wv

    zero = jnp.zeros((16,), jnp.float32)
    nvec = jnp.where(wid == NW - 1, ITERS_LAST, ITERS_FULL)
    num, den = lax.fori_loop(0, nvec, body, (zero, zero))

    partvm[pl.ds(0, 16)] = num
    partvm[pl.ds(16, 16)] = den
    pltpu.sync_copy(partvm, shared.at[s])
    plsc.subcore_barrier()

    @pl.when(s == 0)
    def _():
        pltpu.sync_copy(shared, allvm)
        tn = jnp.zeros((16,), jnp.float32)
        td = jnp.zeros((16,), jnp.float32)
        for s2 in range(NS):
            tn = tn + allvm[s2, pl.ds(0, 16)]
            td = td + allvm[s2, pl.ds(16, 16)]
        num_s = jnp.sum(tn)
        den_s = jnp.sum(td)
        outvm[...] = jnp.where(iota == 0, num_s,
                               jnp.where(iota == 1, den_s, 0.0))
        pltpu.sync_copy(outvm, out_hbm.at[c])


@jax.jit
def _sc_loss(lf, lab, wpad):
    mesh = plsc.VectorSubcoreMesh(core_axis_name="c", subcore_axis_name="s")
    run = pl.kernel(
        _sc_loss_kernel,
        out_type=jax.ShapeDtypeStruct((NC, 16), jnp.float32),
        mesh=mesh,
        scratch_types=[
            pltpu.VMEM((2 * ROWS_FULL,), jnp.float32),
            pltpu.VMEM((ROWS_FULL,), jnp.int32),
            pltpu.VMEM((16,), jnp.float32),
            pltpu.VMEM((32,), jnp.float32),
            pltpu.VMEM((NS, 32), jnp.float32),
            pltpu.VMEM((16,), jnp.float32),
            pltpu.VMEM_SHARED((NS, 32), jnp.float32),
        ],
        compiler_params=pltpu.CompilerParams(needs_layout_passes=False),
    )
    return run(lf, lab, wpad)


def kernel(logits, labels, weight):
    lf = logits.reshape(-1)
    lab = labels.astype(jnp.int32)
    wpad = jnp.pad(weight.astype(jnp.float32), (0, 14))
    out = _sc_loss(lf, lab, wpad)
    num = out[0, 0] + out[1, 0]
    den = out[0, 1] + out[1, 1]
    return num / den
